# Initial kernel scaffold; baseline (speedup 1.0000x reference)
import jax
import jax.numpy as jnp
from jax.experimental import pallas as pl


def kernel(token, table):
    raise NotImplementedError("write your pallas kernel here")


# speedup vs baseline: 1.4228x; 1.4228x over previous
import functools
import math

import jax
import jax.numpy as jnp
from jax import lax
from jax.experimental import pallas as pl
from jax.experimental.pallas import tpu as pltpu
from jax.experimental.pallas import tpu_sc as plsc

_NUM_CORES = 2
_NUM_SUBCORES = 16
_NUM_WORKERS = _NUM_CORES * _NUM_SUBCORES
_LANES = 16


def _make_sc_gather(n_tokens: int, vocab: int, d_model: int):
  assert n_tokens % _NUM_WORKERS == 0
  per_worker = n_tokens // _NUM_WORKERS
  chunk = 32
  while per_worker % chunk:
    chunk //= 2
  n_chunks = per_worker // chunk
  vecs_per_row = d_model // _LANES
  scale = jnp.float32(math.sqrt(d_model))

  mesh = plsc.VectorSubcoreMesh(core_axis_name="c", subcore_axis_name="s")

  @functools.partial(
      pl.kernel,
      mesh=mesh,
      out_type=jax.ShapeDtypeStruct((n_tokens, d_model), jnp.float32),
      scratch_types=[
          pltpu.VMEM((per_worker,), jnp.int32),
          pltpu.VMEM((chunk, d_model), jnp.float32),
          pltpu.VMEM((chunk, d_model), jnp.float32),
          pltpu.SemaphoreType.DMA,
          pltpu.SemaphoreType.DMA,
      ],
  )
  def gather_kernel(tok_hbm, tab_hbm, out_hbm, idx_v, rows0, rows1,
                    sem0, sem1):
    wid = lax.axis_index("s") * _NUM_CORES + lax.axis_index("c")
    base = wid * per_worker

    pltpu.sync_copy(tok_hbm.at[pl.ds(base, per_worker)], idx_v)

    bufs = (rows0, rows1)
    sems = (sem0, sem1)
    copies = [None, None]

    def start_gather(g):
      b = g % 2
      copies[b] = pltpu.async_copy(
          tab_hbm.at[idx_v.at[pl.ds(g * chunk, chunk)]], bufs[b], sems[b])

    start_gather(0)
    for g in range(n_chunks):
      b = g % 2
      copies[b].wait()
      if g + 1 < n_chunks:
        start_gather(g + 1)

      cur = bufs[b]

      @plsc.parallel_loop(0, chunk)
      def _(r):
        for c in range(vecs_per_row):
          sl = pl.ds(c * _LANES, _LANES)
          cur[r, sl] = cur[r, sl] * scale

      pltpu.sync_copy(cur, out_hbm.at[pl.ds(base + g * chunk, chunk)])

  return gather_kernel


def kernel(token, table):
  vocab, d_model = table.shape
  n_tokens = token.size
  tok_flat = token.reshape((n_tokens,)).astype(jnp.int32)
  out = _make_sc_gather(n_tokens, vocab, d_model)(tok_flat, table)
  return out.reshape(token.shape + (d_model,))


# speedup vs baseline: 1.4513x; 1.0200x over previous
import functools
import math

import jax
import jax.numpy as jnp
from jax import lax
from jax.experimental import pallas as pl
from jax.experimental.pallas import tpu as pltpu
from jax.experimental.pallas import tpu_sc as plsc

_NUM_CORES = 2
_NUM_SUBCORES = 16
_NUM_WORKERS = _NUM_CORES * _NUM_SUBCORES
_LANES = 16
_NBUF = 3


def _make_sc_gather(n_tokens: int, vocab: int, d_model: int):
  assert n_tokens % _NUM_WORKERS == 0
  per_worker = n_tokens // _NUM_WORKERS
  chunk = 32
  while per_worker % chunk:
    chunk //= 2
  n_chunks = per_worker // chunk
  vecs_per_row = d_model // _LANES
  scale = jnp.float32(math.sqrt(d_model))

  mesh = plsc.VectorSubcoreMesh(core_axis_name="c", subcore_axis_name="s")

  @functools.partial(
      pl.kernel,
      mesh=mesh,
      out_type=jax.ShapeDtypeStruct((n_tokens, d_model), jnp.float32),
      scratch_types=[
          pltpu.VMEM((per_worker,), jnp.int32),
          *([pltpu.VMEM((chunk, d_model), jnp.float32)] * _NBUF),
          *([pltpu.SemaphoreType.DMA] * (2 * _NBUF)),
      ],
  )
  def gather_kernel(tok_hbm, tab_hbm, out_hbm, idx_v, *bufs_and_sems):
    bufs = bufs_and_sems[:_NBUF]
    gsem = bufs_and_sems[_NBUF:2 * _NBUF]
    ssem = bufs_and_sems[2 * _NBUF:]

    wid = lax.axis_index("s") * _NUM_CORES + lax.axis_index("c")
    base = wid * per_worker

    pltpu.sync_copy(tok_hbm.at[pl.ds(base, per_worker)], idx_v)

    gat = [None] * _NBUF
    scat = {}

    def start_gather(g):
      b = g % _NBUF
      gat[b] = pltpu.async_copy(
          tab_hbm.at[idx_v.at[pl.ds(g * chunk, chunk)]], bufs[b], gsem[b])

    for j in range(min(_NBUF - 1, n_chunks)):
      start_gather(j)

    waited = set()
    for g in range(n_chunks):
      b = g % _NBUF
      ng = g + _NBUF - 1
      if ng < n_chunks:
        prev = ng - _NBUF
        if prev >= 0:
          scat[prev].wait()
          waited.add(prev)
        start_gather(ng)

      gat[b].wait()
      cur = bufs[b]

      @plsc.parallel_loop(0, chunk)
      def _(r):
        for c in range(vecs_per_row):
          sl = pl.ds(c * _LANES, _LANES)
          cur[r, sl] = cur[r, sl] * scale

      scat[g] = pltpu.async_copy(
          cur, out_hbm.at[pl.ds(base + g * chunk, chunk)], ssem[b])

    for g in range(n_chunks):
      if g not in waited:
        scat[g].wait()

  return gather_kernel


def kernel(token, table):
  vocab, d_model = table.shape
  n_tokens = token.size
  tok_flat = token.reshape((n_tokens,)).astype(jnp.int32)
  out = _make_sc_gather(n_tokens, vocab, d_model)(tok_flat, table)
  return out.reshape(token.shape + (d_model,))


# speedup vs baseline: 1.5918x; 1.0968x over previous
import functools
import math

import jax
import jax.numpy as jnp
from jax import lax
from jax.experimental import pallas as pl
from jax.experimental.pallas import tpu as pltpu
from jax.experimental.pallas import tpu_sc as plsc

_NUM_CORES = 2
_NUM_SUBCORES = 16
_NUM_WORKERS = _NUM_CORES * _NUM_SUBCORES
_LANES = 16
_NBUF = 6


def _make_sc_gather(n_tokens: int, vocab: int, d_model: int):
  assert n_tokens % _NUM_WORKERS == 0
  per_worker = n_tokens // _NUM_WORKERS
  chunk = 16
  while per_worker % chunk:
    chunk //= 2
  n_chunks = per_worker // chunk
  vecs_per_row = d_model // _LANES
  scale = jnp.float32(math.sqrt(d_model))

  mesh = plsc.VectorSubcoreMesh(core_axis_name="c", subcore_axis_name="s")

  @functools.partial(
      pl.kernel,
      mesh=mesh,
      out_type=jax.ShapeDtypeStruct((n_tokens, d_model), jnp.float32),
      scratch_types=[
          pltpu.VMEM((per_worker,), jnp.int32),
          *([pltpu.VMEM((chunk, d_model), jnp.float32)] * _NBUF),
          *([pltpu.SemaphoreType.DMA] * (2 * _NBUF)),
      ],
  )
  def gather_kernel(tok_hbm, tab_hbm, out_hbm, idx_v, *bufs_and_sems):
    bufs = bufs_and_sems[:_NBUF]
    gsem = bufs_and_sems[_NBUF:2 * _NBUF]
    ssem = bufs_and_sems[2 * _NBUF:]

    wid = lax.axis_index("s") * _NUM_CORES + lax.axis_index("c")
    base = wid * per_worker

    pltpu.sync_copy(tok_hbm.at[pl.ds(base, per_worker)], idx_v)

    gat = [None] * _NBUF
    scat = {}

    def start_gather(g):
      b = g % _NBUF
      gat[b] = pltpu.async_copy(
          tab_hbm.at[idx_v.at[pl.ds(g * chunk, chunk)]], bufs[b], gsem[b])

    prime = min(_NBUF - 2, n_chunks)
    for j in range(prime):
      start_gather(j)

    waited = set()
    for g in range(n_chunks):
      b = g % _NBUF
      ng = g + prime
      if ng < n_chunks:
        prev = ng - _NBUF
        if prev >= 0:
          scat[prev].wait()
          waited.add(prev)
        start_gather(ng)

      gat[b].wait()
      cur = bufs[b]

      @plsc.parallel_loop(0, chunk * vecs_per_row, unroll=8)
      def _(i):
        r = i // vecs_per_row
        sl = pl.ds((i % vecs_per_row) * _LANES, _LANES)
        cur[r, sl] = cur[r, sl] * scale

      scat[g] = pltpu.async_copy(
          cur, out_hbm.at[pl.ds(base + g * chunk, chunk)], ssem[b])

    for g in range(n_chunks):
      if g not in waited:
        scat[g].wait()

  return gather_kernel


def kernel(token, table):
  vocab, d_model = table.shape
  n_tokens = token.size
  tok_flat = token.reshape((n_tokens,)).astype(jnp.int32)
  out = _make_sc_gather(n_tokens, vocab, d_model)(tok_flat, table)
  return out.reshape(token.shape + (d_model,))


# speedup vs baseline: 1.6010x; 1.0058x over previous
import functools
import math

import jax
import jax.numpy as jnp
from jax import lax
from jax.experimental import pallas as pl
from jax.experimental.pallas import tpu as pltpu
from jax.experimental.pallas import tpu_sc as plsc

_NUM_CORES = 2
_NUM_SUBCORES = 16
_NUM_WORKERS = _NUM_CORES * _NUM_SUBCORES
_LANES = 16
_NBUF = 7


def _make_sc_gather(n_tokens: int, vocab: int, d_model: int):
  assert n_tokens % _NUM_WORKERS == 0
  per_worker = n_tokens // _NUM_WORKERS
  chunk = 16
  while per_worker % chunk:
    chunk //= 2
  n_chunks = per_worker // chunk
  vecs_per_row = d_model // _LANES
  scale = jnp.float32(math.sqrt(d_model))

  mesh = plsc.VectorSubcoreMesh(core_axis_name="c", subcore_axis_name="s")

  @functools.partial(
      pl.kernel,
      mesh=mesh,
      out_type=jax.ShapeDtypeStruct((n_tokens, d_model), jnp.float32),
      scratch_types=[
          pltpu.VMEM((per_worker,), jnp.int32),
          *([pltpu.VMEM((chunk, d_model), jnp.float32)] * _NBUF),
          *([pltpu.SemaphoreType.DMA] * (2 * _NBUF)),
      ],
  )
  def gather_kernel(tok_hbm, tab_hbm, out_hbm, idx_v, *bufs_and_sems):
    bufs = bufs_and_sems[:_NBUF]
    gsem = bufs_and_sems[_NBUF:2 * _NBUF]
    ssem = bufs_and_sems[2 * _NBUF:]

    wid = lax.axis_index("s") * _NUM_CORES + lax.axis_index("c")
    base = wid * per_worker

    pltpu.sync_copy(tok_hbm.at[pl.ds(base, per_worker)], idx_v)

    gat = [None] * _NBUF
    scat = {}

    def start_gather(g):
      b = g % _NBUF
      gat[b] = pltpu.async_copy(
          tab_hbm.at[idx_v.at[pl.ds(g * chunk, chunk)]], bufs[b], gsem[b])

    prime = min(_NBUF - 2, n_chunks)
    for j in range(prime):
      start_gather(j)

    waited = set()
    for g in range(n_chunks):
      b = g % _NBUF
      ng = g + prime
      if ng < n_chunks:
        prev = ng - _NBUF
        if prev >= 0:
          scat[prev].wait()
          waited.add(prev)
        start_gather(ng)

      gat[b].wait()
      cur = bufs[b]

      @plsc.parallel_loop(0, chunk * vecs_per_row, unroll=8)
      def _(i):
        r = i // vecs_per_row
        sl = pl.ds((i % vecs_per_row) * _LANES, _LANES)
        cur[r, sl] = cur[r, sl] * scale

      scat[g] = pltpu.async_copy(
          cur, out_hbm.at[pl.ds(base + g * chunk, chunk)], ssem[b])

    for g in range(n_chunks):
      if g not in waited:
        scat[g].wait()

  return gather_kernel


def kernel(token, table):
  vocab, d_model = table.shape
  n_tokens = token.size
  tok_flat = token.reshape((n_tokens,)).astype(jnp.int32)
  out = _make_sc_gather(n_tokens, vocab, d_model)(tok_flat, table)
  return out.reshape(token.shape + (d_model,))
